# baseline (speedup 1.0000x reference)
import functools

import jax
import jax.numpy as jnp
from jax import lax
from jax.experimental import pallas as pl
from jax.experimental.pallas import tpu as pltpu
from jax.experimental.pallas import tpu_sc as plsc

VOCAB = 1000000
EMBED_DIM = 64
BATCH = 4096
P_LEN = 50

NUM_IDX = BATCH * P_LEN
NUM_WORKERS = 32
PER_WORKER = NUM_IDX // NUM_WORKERS

TBLK = 512
TGRID = (VOCAB + TBLK - 1) // TBLK
TROWS = TGRID * (TBLK // 2)

CHUNK = 64
PAIRS = 2 * CHUNK
NCHUNK = PER_WORKER // CHUNK
NBUF = 5
LOOKAHEAD = 2


def _tc_transpose(wv_t):
    def body(i_ref, o_ref):
        t = i_ref[...]
        a = t.reshape(EMBED_DIM, TBLK // 2, 2)
        o_ref[:, 0:EMBED_DIM] = jnp.transpose(a[:, :, 0], (1, 0))
        o_ref[:, EMBED_DIM:2 * EMBED_DIM] = jnp.transpose(a[:, :, 1], (1, 0))

    return pl.pallas_call(
        body,
        grid=(TGRID,),
        in_specs=[pl.BlockSpec((EMBED_DIM, TBLK), lambda g: (0, g))],
        out_specs=pl.BlockSpec((TBLK // 2, 128), lambda g: (g, 0)),
        out_shape=jax.ShapeDtypeStruct((TROWS, 128), jnp.float32),
    )(wv_t)


def _build_gather():
    mesh = plsc.VectorSubcoreMesh(core_axis_name="c", subcore_axis_name="s")

    scratch = [pltpu.VMEM((NCHUNK, PAIRS), jnp.int32)]
    scratch += [pltpu.VMEM((PAIRS, 32), jnp.float32) for _ in range(NBUF)]
    scratch += [pltpu.SemaphoreType.DMA for _ in range(2 * NBUF)]

    @functools.partial(
        pl.kernel,
        mesh=mesh,
        compiler_params=pltpu.CompilerParams(use_tc_tiling_on_sc=False),
        out_type=jax.ShapeDtypeStruct((2 * NUM_IDX, 32), jnp.float32),
        scratch_types=scratch,
    )
    def gather_kernel(table_hbm, idx_hbm, out_hbm, idx_v, *bufs_and_sems):
        rows = bufs_and_sems[:NBUF]
        sem_g = bufs_and_sems[NBUF:2 * NBUF]
        sem_w = bufs_and_sems[2 * NBUF:]

        wid = lax.axis_index("s") * 2 + lax.axis_index("c")
        base = wid * PER_WORKER
        pltpu.sync_copy(idx_hbm.at[wid], idx_v)

        def fire_gather(f, bf):
            pltpu.async_copy(table_hbm.at[idx_v.at[f]], rows[bf], sem_g[bf])

        def wait_gather(bf):
            pltpu.make_async_copy(
                table_hbm.at[idx_v.at[0]], rows[bf], sem_g[bf]
            ).wait()

        def out_slice(g):
            return out_hbm.at[pl.ds(2 * (base + g * CHUNK), PAIRS)]

        def fire_write(g, b):
            pltpu.async_copy(rows[b], out_slice(g), sem_w[b])

        def wait_write(b):
            pltpu.make_async_copy(rows[b], out_slice(0), sem_w[b]).wait()

        for b in range(LOOKAHEAD):
            fire_gather(b, b)

        def body(t, carry):
            for b in range(NBUF):
                g = t * NBUF + b
                f = g + LOOKAHEAD
                bf = (b + LOOKAHEAD) % NBUF

                @pl.when(f < NCHUNK)
                def _():
                    @pl.when(f >= NBUF)
                    def _():
                        wait_write(bf)

                    fire_gather(f, bf)

                wait_gather(b)
                fire_write(g, b)
            return carry

        lax.fori_loop(0, NCHUNK // NBUF, body, 0)

        for b in range(NBUF):
            wait_write(b)

    return gather_kernel


_gather = _build_gather()


def kernel(x, pw_idxs, qw_idxs, p_mask, q_mask, word_vectors):
    tbl2 = _tc_transpose(word_vectors.T)
    view = tbl2.reshape(4 * TROWS, 32)
    idx = x.astype(jnp.int32).reshape(-1)
    idx32 = (2 * idx[:, None] + jnp.arange(2, dtype=jnp.int32)[None, :]).reshape(
        NUM_WORKERS, NCHUNK, PAIRS
    )
    out = _gather(view, idx32)
    return out.reshape(BATCH, P_LEN, EMBED_DIM)


# speedup vs baseline: 16.7705x; 16.7705x over previous
import functools

import jax
import jax.numpy as jnp
from jax import lax
from jax.experimental import pallas as pl
from jax.experimental.pallas import tpu as pltpu
from jax.experimental.pallas import tpu_sc as plsc

VOCAB = 1000000
EMBED_DIM = 64
BATCH = 4096
P_LEN = 50

NUM_IDX = BATCH * P_LEN
NUM_WORKERS = 32
PER_WORKER = NUM_IDX // NUM_WORKERS

TBLK = 2048
HBLK = TBLK // 2
TGRID = (VOCAB + TBLK - 1) // TBLK
TROWS = TGRID * HBLK

CHUNK = 64
PAIRS = 2 * CHUNK
NCHUNK = PER_WORKER // CHUNK
NBUF = 5
LOOKAHEAD = 2


def _tc_transpose(wv_t):
    def body(i_ref, o_ref):
        x = i_ref[...]
        eye = jnp.eye(EMBED_DIM, dtype=jnp.float32)
        dims = (((0,), (0,)), ((), ()))
        o_ref[:, 0:EMBED_DIM] = jax.lax.dot_general(
            x[:, 0:HBLK], eye, dims, precision=jax.lax.Precision.HIGHEST
        )
        o_ref[:, EMBED_DIM:2 * EMBED_DIM] = jax.lax.dot_general(
            x[:, HBLK:TBLK], eye, dims, precision=jax.lax.Precision.HIGHEST
        )

    return pl.pallas_call(
        body,
        grid=(TGRID,),
        in_specs=[pl.BlockSpec((EMBED_DIM, TBLK), lambda g: (0, g))],
        out_specs=pl.BlockSpec((HBLK, 128), lambda g: (g, 0)),
        out_shape=jax.ShapeDtypeStruct((TROWS, 128), jnp.float32),
    )(wv_t)


def _build_gather():
    mesh = plsc.VectorSubcoreMesh(core_axis_name="c", subcore_axis_name="s")

    scratch = [pltpu.VMEM((NCHUNK, PAIRS), jnp.int32)]
    scratch += [pltpu.VMEM((PAIRS, 32), jnp.float32) for _ in range(NBUF)]
    scratch += [pltpu.SemaphoreType.DMA for _ in range(2 * NBUF)]

    @functools.partial(
        pl.kernel,
        mesh=mesh,
        compiler_params=pltpu.CompilerParams(use_tc_tiling_on_sc=False),
        out_type=jax.ShapeDtypeStruct((2 * NUM_IDX, 32), jnp.float32),
        scratch_types=scratch,
    )
    def gather_kernel(table_hbm, idx_hbm, out_hbm, idx_v, *bufs_and_sems):
        rows = bufs_and_sems[:NBUF]
        sem_g = bufs_and_sems[NBUF:2 * NBUF]
        sem_w = bufs_and_sems[2 * NBUF:]

        wid = lax.axis_index("s") * 2 + lax.axis_index("c")
        base = wid * PER_WORKER
        pltpu.sync_copy(idx_hbm.at[wid], idx_v)

        def fire_gather(f, bf):
            pltpu.async_copy(table_hbm.at[idx_v.at[f]], rows[bf], sem_g[bf])

        def wait_gather(bf):
            pltpu.make_async_copy(
                table_hbm.at[idx_v.at[0]], rows[bf], sem_g[bf]
            ).wait()

        def out_slice(g):
            return out_hbm.at[pl.ds(2 * (base + g * CHUNK), PAIRS)]

        def fire_write(g, b):
            pltpu.async_copy(rows[b], out_slice(g), sem_w[b])

        def wait_write(b):
            pltpu.make_async_copy(rows[b], out_slice(0), sem_w[b]).wait()

        for b in range(LOOKAHEAD):
            fire_gather(b, b)

        def body(t, carry):
            for b in range(NBUF):
                g = t * NBUF + b
                f = g + LOOKAHEAD
                bf = (b + LOOKAHEAD) % NBUF

                @pl.when(f < NCHUNK)
                def _():
                    @pl.when(f >= NBUF)
                    def _():
                        wait_write(bf)

                    fire_gather(f, bf)

                wait_gather(b)
                fire_write(g, b)
            return carry

        lax.fori_loop(0, NCHUNK // NBUF, body, 0)

        for b in range(NBUF):
            wait_write(b)

    return gather_kernel


_gather = _build_gather()


def kernel(x, pw_idxs, qw_idxs, p_mask, q_mask, word_vectors):
    tbl2 = _tc_transpose(word_vectors.T)
    view = tbl2.reshape(4 * TROWS, 32)
    idx = x.astype(jnp.int32).reshape(-1)
    l = idx % TBLK
    row = HBLK * (idx // TBLK) + (l % HBLK)
    half = l // HBLK
    base32 = 4 * row + 2 * half
    idx32 = (base32[:, None] + jnp.arange(2, dtype=jnp.int32)[None, :]).reshape(
        NUM_WORKERS, NCHUNK, PAIRS
    )
    out = _gather(view, idx32)
    return out.reshape(BATCH, P_LEN, EMBED_DIM)


# speedup vs baseline: 25.0814x; 1.4956x over previous
import functools

import jax
import jax.numpy as jnp
from jax import lax
from jax.experimental import pallas as pl
from jax.experimental.pallas import tpu as pltpu
from jax.experimental.pallas import tpu_sc as plsc

VOCAB = 1000000
EMBED_DIM = 64
BATCH = 4096
P_LEN = 50

NUM_IDX = BATCH * P_LEN
NUM_WORKERS = 32
PER_WORKER = NUM_IDX // NUM_WORKERS

TBLK = 4096
QBLK = TBLK // 4
TGRID = (VOCAB + TBLK - 1) // TBLK
TROWS = TGRID * 2 * QBLK

CHUNK = 64
PAIRS = 2 * CHUNK
NCHUNK = PER_WORKER // CHUNK
NBUF = 5
LOOKAHEAD = 2


def _tc_transpose(wv_t):
    def body(i_ref, o_ref):
        x = i_ref[...]
        x4 = jnp.concatenate(
            [x[:, q * QBLK:(q + 1) * QBLK] for q in range(4)], axis=0
        )
        eye = jnp.eye(256, dtype=jnp.float32)
        t = jax.lax.dot_general(
            x4, eye, (((0,), (0,)), ((), ())),
            precision=jax.lax.Precision.HIGHEST,
        )
        o_ref[0:QBLK, :] = t[:, 0:128]
        o_ref[QBLK:2 * QBLK, :] = t[:, 128:256]

    return pl.pallas_call(
        body,
        grid=(TGRID,),
        in_specs=[pl.BlockSpec((EMBED_DIM, TBLK), lambda g: (0, g))],
        out_specs=pl.BlockSpec((2 * QBLK, 128), lambda g: (g, 0)),
        out_shape=jax.ShapeDtypeStruct((TROWS, 128), jnp.float32),
    )(wv_t)


def _build_gather():
    mesh = plsc.VectorSubcoreMesh(core_axis_name="c", subcore_axis_name="s")

    scratch = [pltpu.VMEM((NCHUNK, PAIRS), jnp.int32)]
    scratch += [pltpu.VMEM((PAIRS, 32), jnp.float32) for _ in range(NBUF)]
    scratch += [pltpu.SemaphoreType.DMA for _ in range(2 * NBUF)]

    @functools.partial(
        pl.kernel,
        mesh=mesh,
        compiler_params=pltpu.CompilerParams(use_tc_tiling_on_sc=False),
        out_type=jax.ShapeDtypeStruct((2 * NUM_IDX, 32), jnp.float32),
        scratch_types=scratch,
    )
    def gather_kernel(table_hbm, idx_hbm, out_hbm, idx_v, *bufs_and_sems):
        rows = bufs_and_sems[:NBUF]
        sem_g = bufs_and_sems[NBUF:2 * NBUF]
        sem_w = bufs_and_sems[2 * NBUF:]

        wid = lax.axis_index("s") * 2 + lax.axis_index("c")
        base = wid * PER_WORKER
        pltpu.sync_copy(idx_hbm.at[wid], idx_v)

        def fire_gather(f, bf):
            pltpu.async_copy(table_hbm.at[idx_v.at[f]], rows[bf], sem_g[bf])

        def wait_gather(bf):
            pltpu.make_async_copy(
                table_hbm.at[idx_v.at[0]], rows[bf], sem_g[bf]
            ).wait()

        def out_slice(g):
            return out_hbm.at[pl.ds(2 * (base + g * CHUNK), PAIRS)]

        def fire_write(g, b):
            pltpu.async_copy(rows[b], out_slice(g), sem_w[b])

        def wait_write(b):
            pltpu.make_async_copy(rows[b], out_slice(0), sem_w[b]).wait()

        for b in range(LOOKAHEAD):
            fire_gather(b, b)

        def body(t, carry):
            for b in range(NBUF):
                g = t * NBUF + b
                f = g + LOOKAHEAD
                bf = (b + LOOKAHEAD) % NBUF

                @pl.when(f < NCHUNK)
                def _():
                    @pl.when(f >= NBUF)
                    def _():
                        wait_write(bf)

                    fire_gather(f, bf)

                wait_gather(b)
                fire_write(g, b)
            return carry

        lax.fori_loop(0, NCHUNK // NBUF, body, 0)

        for b in range(NBUF):
            wait_write(b)

    return gather_kernel


_gather = _build_gather()


def kernel(x, pw_idxs, qw_idxs, p_mask, q_mask, word_vectors):
    tbl2 = _tc_transpose(word_vectors.T)
    view = tbl2.reshape(4 * TROWS, 32)
    idx = x.astype(jnp.int32).reshape(-1)
    l = idx % TBLK
    q = l // QBLK
    row = 2 * QBLK * (idx // TBLK) + QBLK * (q // 2) + (l % QBLK)
    base32 = 4 * row + 2 * (q % 2)
    idx32 = (base32[:, None] + jnp.arange(2, dtype=jnp.int32)[None, :]).reshape(
        NUM_WORKERS, NCHUNK, PAIRS
    )
    out = _gather(view, idx32)
    return out.reshape(BATCH, P_LEN, EMBED_DIM)


# speedup vs baseline: 29.2191x; 1.1650x over previous
import functools

import jax
import jax.numpy as jnp
from jax import lax
from jax.experimental import pallas as pl
from jax.experimental.pallas import tpu as pltpu
from jax.experimental.pallas import tpu_sc as plsc

VOCAB = 1000000
EMBED_DIM = 64
BATCH = 4096
P_LEN = 50

NUM_IDX = BATCH * P_LEN
NUM_WORKERS = 32
PER_WORKER = NUM_IDX // NUM_WORKERS

TBLK = 8192
QBLK = TBLK // 4
TGRID = (VOCAB + TBLK - 1) // TBLK
TROWS = TGRID * 2 * QBLK

CHUNK = 64
PAIRS = 2 * CHUNK
NCHUNK = PER_WORKER // CHUNK
NBUF = 5
LOOKAHEAD = 2


def _tc_transpose(wv_t):
    def body(i_ref, o_ref):
        x = i_ref[...]
        x4 = jnp.concatenate(
            [x[:, q * QBLK:(q + 1) * QBLK] for q in range(4)], axis=0
        )
        eye = jnp.eye(256, dtype=jnp.float32)
        t = jax.lax.dot_general(
            x4, eye, (((0,), (0,)), ((), ())),
            precision=jax.lax.Precision.HIGHEST,
        )
        o_ref[0:QBLK, :] = t[:, 0:128]
        o_ref[QBLK:2 * QBLK, :] = t[:, 128:256]

    return pl.pallas_call(
        body,
        grid=(TGRID,),
        in_specs=[pl.BlockSpec((EMBED_DIM, TBLK), lambda g: (0, g))],
        out_specs=pl.BlockSpec((2 * QBLK, 128), lambda g: (g, 0)),
        out_shape=jax.ShapeDtypeStruct((TROWS, 128), jnp.float32),
    )(wv_t)


def _build_gather():
    mesh = plsc.VectorSubcoreMesh(core_axis_name="c", subcore_axis_name="s")

    scratch = [pltpu.VMEM((NCHUNK, PAIRS), jnp.int32)]
    scratch += [pltpu.VMEM((PAIRS, 32), jnp.float32) for _ in range(NBUF)]
    scratch += [pltpu.SemaphoreType.DMA for _ in range(2 * NBUF)]

    @functools.partial(
        pl.kernel,
        mesh=mesh,
        compiler_params=pltpu.CompilerParams(use_tc_tiling_on_sc=False),
        out_type=jax.ShapeDtypeStruct((2 * NUM_IDX, 32), jnp.float32),
        scratch_types=scratch,
    )
    def gather_kernel(table_hbm, idx_hbm, out_hbm, idx_v, *bufs_and_sems):
        rows = bufs_and_sems[:NBUF]
        sem_g = bufs_and_sems[NBUF:2 * NBUF]
        sem_w = bufs_and_sems[2 * NBUF:]

        wid = lax.axis_index("s") * 2 + lax.axis_index("c")
        base = wid * PER_WORKER
        pltpu.sync_copy(idx_hbm.at[wid], idx_v)

        def fire_gather(f, bf):
            pltpu.async_copy(table_hbm.at[idx_v.at[f]], rows[bf], sem_g[bf])

        def wait_gather(bf):
            pltpu.make_async_copy(
                table_hbm.at[idx_v.at[0]], rows[bf], sem_g[bf]
            ).wait()

        def out_slice(g):
            return out_hbm.at[pl.ds(2 * (base + g * CHUNK), PAIRS)]

        def fire_write(g, b):
            pltpu.async_copy(rows[b], out_slice(g), sem_w[b])

        def wait_write(b):
            pltpu.make_async_copy(rows[b], out_slice(0), sem_w[b]).wait()

        for b in range(LOOKAHEAD):
            fire_gather(b, b)

        def body(t, carry):
            for b in range(NBUF):
                g = t * NBUF + b
                f = g + LOOKAHEAD
                bf = (b + LOOKAHEAD) % NBUF

                @pl.when(f < NCHUNK)
                def _():
                    @pl.when(f >= NBUF)
                    def _():
                        wait_write(bf)

                    fire_gather(f, bf)

                wait_gather(b)
                fire_write(g, b)
            return carry

        lax.fori_loop(0, NCHUNK // NBUF, body, 0)

        for b in range(NBUF):
            wait_write(b)

    return gather_kernel


_gather = _build_gather()


def kernel(x, pw_idxs, qw_idxs, p_mask, q_mask, word_vectors):
    tbl2 = _tc_transpose(word_vectors.T)
    view = tbl2.reshape(4 * TROWS, 32)
    idx = x.astype(jnp.int32).reshape(-1)
    l = idx % TBLK
    q = l // QBLK
    row = 2 * QBLK * (idx // TBLK) + QBLK * (q // 2) + (l % QBLK)
    base32 = 4 * row + 2 * (q % 2)
    idx32 = (base32[:, None] + jnp.arange(2, dtype=jnp.int32)[None, :]).reshape(
        NUM_WORKERS, NCHUNK, PAIRS
    )
    out = _gather(view, idx32)
    return out.reshape(BATCH, P_LEN, EMBED_DIM)


# speedup vs baseline: 31.6149x; 1.0820x over previous
import functools

import jax
import jax.numpy as jnp
from jax import lax
from jax.experimental import pallas as pl
from jax.experimental.pallas import tpu as pltpu
from jax.experimental.pallas import tpu_sc as plsc

VOCAB = 1000000
EMBED_DIM = 64
BATCH = 4096
P_LEN = 50

NUM_IDX = BATCH * P_LEN
NUM_WORKERS = 32
PER_WORKER = NUM_IDX // NUM_WORKERS

TBLK = 16384
QBLK = TBLK // 4
TGRID = (VOCAB + TBLK - 1) // TBLK
TROWS = TGRID * 2 * QBLK

CHUNK = 64
PAIRS = 2 * CHUNK
NCHUNK = PER_WORKER // CHUNK
NBUF = 5
LOOKAHEAD = 2


def _tc_transpose(wv_t):
    def body(i_ref, o_ref):
        x = i_ref[...]
        x4 = jnp.concatenate(
            [x[:, q * QBLK:(q + 1) * QBLK] for q in range(4)], axis=0
        )
        eye = jnp.eye(256, dtype=jnp.float32)
        t = jax.lax.dot_general(
            x4, eye, (((0,), (0,)), ((), ())),
            precision=jax.lax.Precision.HIGHEST,
        )
        o_ref[0:QBLK, :] = t[:, 0:128]
        o_ref[QBLK:2 * QBLK, :] = t[:, 128:256]

    return pl.pallas_call(
        body,
        grid=(TGRID,),
        in_specs=[pl.BlockSpec((EMBED_DIM, TBLK), lambda g: (0, g))],
        out_specs=pl.BlockSpec((2 * QBLK, 128), lambda g: (g, 0)),
        out_shape=jax.ShapeDtypeStruct((TROWS, 128), jnp.float32),
    )(wv_t)


def _build_gather():
    mesh = plsc.VectorSubcoreMesh(core_axis_name="c", subcore_axis_name="s")

    scratch = [pltpu.VMEM((NCHUNK, PAIRS), jnp.int32)]
    scratch += [pltpu.VMEM((PAIRS, 32), jnp.float32) for _ in range(NBUF)]
    scratch += [pltpu.SemaphoreType.DMA for _ in range(2 * NBUF)]

    @functools.partial(
        pl.kernel,
        mesh=mesh,
        compiler_params=pltpu.CompilerParams(use_tc_tiling_on_sc=False),
        out_type=jax.ShapeDtypeStruct((2 * NUM_IDX, 32), jnp.float32),
        scratch_types=scratch,
    )
    def gather_kernel(table_hbm, idx_hbm, out_hbm, idx_v, *bufs_and_sems):
        rows = bufs_and_sems[:NBUF]
        sem_g = bufs_and_sems[NBUF:2 * NBUF]
        sem_w = bufs_and_sems[2 * NBUF:]

        wid = lax.axis_index("s") * 2 + lax.axis_index("c")
        base = wid * PER_WORKER
        pltpu.sync_copy(idx_hbm.at[wid], idx_v)

        def fire_gather(f, bf):
            pltpu.async_copy(table_hbm.at[idx_v.at[f]], rows[bf], sem_g[bf])

        def wait_gather(bf):
            pltpu.make_async_copy(
                table_hbm.at[idx_v.at[0]], rows[bf], sem_g[bf]
            ).wait()

        def out_slice(g):
            return out_hbm.at[pl.ds(2 * (base + g * CHUNK), PAIRS)]

        def fire_write(g, b):
            pltpu.async_copy(rows[b], out_slice(g), sem_w[b])

        def wait_write(b):
            pltpu.make_async_copy(rows[b], out_slice(0), sem_w[b]).wait()

        for b in range(LOOKAHEAD):
            fire_gather(b, b)

        def body(t, carry):
            for b in range(NBUF):
                g = t * NBUF + b
                f = g + LOOKAHEAD
                bf = (b + LOOKAHEAD) % NBUF

                @pl.when(f < NCHUNK)
                def _():
                    @pl.when(f >= NBUF)
                    def _():
                        wait_write(bf)

                    fire_gather(f, bf)

                wait_gather(b)
                fire_write(g, b)
            return carry

        lax.fori_loop(0, NCHUNK // NBUF, body, 0)

        for b in range(NBUF):
            wait_write(b)

    return gather_kernel


_gather = _build_gather()


def kernel(x, pw_idxs, qw_idxs, p_mask, q_mask, word_vectors):
    tbl2 = _tc_transpose(word_vectors.T)
    view = tbl2.reshape(4 * TROWS, 32)
    idx = x.astype(jnp.int32).reshape(-1)
    l = idx % TBLK
    q = l // QBLK
    row = 2 * QBLK * (idx // TBLK) + QBLK * (q // 2) + (l % QBLK)
    base32 = 4 * row + 2 * (q % 2)
    idx32 = (base32[:, None] + jnp.arange(2, dtype=jnp.int32)[None, :]).reshape(
        NUM_WORKERS, NCHUNK, PAIRS
    )
    out = _gather(view, idx32)
    return out.reshape(BATCH, P_LEN, EMBED_DIM)
